# Initial kernel scaffold; baseline (speedup 1.0000x reference)
import jax
import jax.numpy as jnp
from jax.experimental import pallas as pl


def kernel(x, edge_index, W1_l, W1_r, b1, W2_l, W2_r, b2):
    raise NotImplementedError("write your pallas kernel here")


# speedup vs baseline: 6.7866x; 6.7866x over previous
import functools

import jax
import jax.numpy as jnp
from jax import lax
from jax.experimental import pallas as pl
from jax.experimental.pallas import tpu as pltpu
from jax.experimental.pallas import tpu_sc as plsc

_N = 10000
_NPAD = 10240
_E = 320000
_D = 128
_NC, _NS, _L = 2, 16, 16
_NW = _NC * _NS
_EPT = _E // _NW
_CH = 80
_NCHUNK = _EPT // _CH
_RPT = _NPAD // _NS
_R = 256
_G = _NPAD // _R


_IBLK = 25


def _segsum_body(with_counts):
  def body(*refs):
    if with_counts:
      (y_hbm, src_hbm, dst_hbm, out_hbm, cnt_hbm,
       src_v, dst_v, rows_v, ones_v, acc_sh, cnt_sh, sem) = refs
    else:
      (y_hbm, src_hbm, dst_hbm, out_hbm,
       src_v, dst_v, rows_v, acc_sh, sem) = refs
    c = lax.axis_index("c")
    s = lax.axis_index("s")
    wid = c * _NS + s

    zf = jnp.zeros((_L,), jnp.float32)

    @pl.loop(0, _CH)
    def _(i):
      for j in range(_D // _L):
        rows_v[i, pl.ds(j * _L, _L)] = zf

    base = s * _RPT
    for k in range(_RPT // _CH):
      pltpu.sync_copy(rows_v, acc_sh.at[pl.ds(base + k * _CH, _CH)])

    if with_counts:
      @pl.loop(0, _CH)
      def _(i):
        ones_v[i, :] = zf

      for k in range(_RPT // _CH):
        pltpu.sync_copy(ones_v, cnt_sh.at[pl.ds(base + k * _CH, _CH)])

      of = jnp.ones((_L,), jnp.float32)

      @pl.loop(0, _CH)
      def _(i):
        ones_v[i, :] = of

    plsc.subcore_barrier()

    @pl.loop(0, _NCHUNK // _IBLK)
    def _(blk):
      pltpu.sync_copy(src_hbm.at[wid, pl.ds(blk * _IBLK, _IBLK)], src_v)
      pltpu.sync_copy(dst_hbm.at[wid, pl.ds(blk * _IBLK, _IBLK)], dst_v)

      @pl.loop(0, _IBLK)
      def _(g):
        pltpu.async_copy(y_hbm.at[src_v.at[g]], rows_v, sem).wait()
        pltpu.sync_copy(rows_v, acc_sh.at[dst_v.at[g]], add=True)
        if with_counts:
          pltpu.sync_copy(ones_v, cnt_sh.at[dst_v.at[g]], add=True)

    plsc.subcore_barrier()

    pltpu.sync_copy(acc_sh.at[pl.ds(base, _RPT)],
                    out_hbm.at[c, pl.ds(base, _RPT)])
    if with_counts:
      pltpu.sync_copy(cnt_sh.at[pl.ds(base, _RPT)],
                      cnt_hbm.at[c, pl.ds(base, _RPT)])
  return body


def _make_segsum(with_counts):
  mesh = plsc.VectorSubcoreMesh(core_axis_name="c", subcore_axis_name="s",
                                num_cores=_NC, num_subcores=_NS)
  out_type = [jax.ShapeDtypeStruct((_NC, _NPAD, _D), jnp.float32)]
  scratch = [
      pltpu.VMEM((_IBLK, _CH), jnp.int32),
      pltpu.VMEM((_IBLK, _CH), jnp.int32),
      pltpu.VMEM((_CH, _D), jnp.float32),
  ]
  if with_counts:
    out_type.append(jax.ShapeDtypeStruct((_NC, _NPAD, _L), jnp.float32))
    scratch.append(pltpu.VMEM((_CH, _L), jnp.float32))
  scratch.append(pltpu.VMEM_SHARED((_NPAD, _D), jnp.float32))
  if with_counts:
    scratch.append(pltpu.VMEM_SHARED((_NPAD, _L), jnp.float32))
  scratch.append(pltpu.SemaphoreType.DMA)
  return pl.kernel(_segsum_body(with_counts), out_type=out_type,
                   mesh=mesh, scratch_types=scratch,
                   compiler_params=pltpu.CompilerParams(
                       use_tc_tiling_on_sc=False))


_seg_cnt = _make_segsum(True)
_seg = _make_segsum(False)


def _pre_body(x_ref, wl_ref, wr_ref, b_ref, y_ref, r_ref):
  xb = x_ref[...]
  y_ref[...] = jnp.dot(xb, wl_ref[...], preferred_element_type=jnp.float32)
  r_ref[...] = (jnp.dot(xb, wr_ref[...], preferred_element_type=jnp.float32)
                + b_ref[...])


_pre = pl.pallas_call(
    _pre_body,
    grid=(_G,),
    in_specs=[
        pl.BlockSpec((_R, _D), lambda i: (i, 0)),
        pl.BlockSpec((_D, _D), lambda i: (0, 0)),
        pl.BlockSpec((_D, _D), lambda i: (0, 0)),
        pl.BlockSpec((1, _D), lambda i: (0, 0)),
    ],
    out_specs=[pl.BlockSpec((_R, _D), lambda i: (i, 0))] * 2,
    out_shape=[jax.ShapeDtypeStruct((_NPAD, _D), jnp.float32)] * 2,
)


def _mid_body(p_ref, cnt_ref, r1_ref, wl_ref, wr_ref, b_ref,
              y_ref, r_ref, inv_ref):
  cnt = cnt_ref[...]
  ctot = cnt[0, 0] + cnt[1, 0]
  inv = 1.0 / jnp.maximum(ctot, 1.0)
  inv_ref[...] = inv[None]
  p = p_ref[...]
  m = (p[0, 0] + p[1, 0]) * inv[:, :1]
  h = jnp.maximum(m + r1_ref[...], 0.0)
  y_ref[...] = jnp.dot(h, wl_ref[...], preferred_element_type=jnp.float32)
  r_ref[...] = (jnp.dot(h, wr_ref[...], preferred_element_type=jnp.float32)
                + b_ref[...])


_mid = pl.pallas_call(
    _mid_body,
    grid=(_G,),
    in_specs=[
        pl.BlockSpec((_NC, 1, _R, _D), lambda i: (0, i, 0, 0)),
        pl.BlockSpec((_NC, 1, _R, _L), lambda i: (0, i, 0, 0)),
        pl.BlockSpec((_R, _D), lambda i: (i, 0)),
        pl.BlockSpec((_D, _D), lambda i: (0, 0)),
        pl.BlockSpec((_D, _D), lambda i: (0, 0)),
        pl.BlockSpec((1, _D), lambda i: (0, 0)),
    ],
    out_specs=[
        pl.BlockSpec((_R, _D), lambda i: (i, 0)),
        pl.BlockSpec((_R, _D), lambda i: (i, 0)),
        pl.BlockSpec((1, _R, _L), lambda i: (i, 0, 0)),
    ],
    out_shape=[
        jax.ShapeDtypeStruct((_NPAD, _D), jnp.float32),
        jax.ShapeDtypeStruct((_NPAD, _D), jnp.float32),
        jax.ShapeDtypeStruct((_G, _R, _L), jnp.float32),
    ],
)


def _post_body(p_ref, inv_ref, r2_ref, o_ref):
  p = p_ref[...]
  inv = inv_ref[...][0]
  o_ref[...] = (p[0, 0] + p[1, 0]) * inv[:, :1] + r2_ref[...]


_post = pl.pallas_call(
    _post_body,
    grid=(_G,),
    in_specs=[
        pl.BlockSpec((_NC, 1, _R, _D), lambda i: (0, i, 0, 0)),
        pl.BlockSpec((1, _R, _L), lambda i: (i, 0, 0)),
        pl.BlockSpec((_R, _D), lambda i: (i, 0)),
    ],
    out_specs=pl.BlockSpec((_R, _D), lambda i: (i, 0)),
    out_shape=jax.ShapeDtypeStruct((_NPAD, _D), jnp.float32),
)


def kernel(x, edge_index, W1_l, W1_r, b1, W2_l, W2_r, b2):
  xp = jnp.zeros((_NPAD, _D), jnp.float32).at[:_N].set(x)
  src = edge_index[0].reshape(_NW, _NCHUNK, _CH)
  dst = edge_index[1].reshape(_NW, _NCHUNK, _CH)
  y1, r1 = _pre(xp, W1_l.T, W1_r.T, b1.reshape(1, _D))
  p, cnt = _seg_cnt(y1, src, dst)
  y2, r2, inv = _mid(p.reshape(_NC, _G, _R, _D), cnt.reshape(_NC, _G, _R, _L),
                     r1, W2_l.T, W2_r.T, b2.reshape(1, _D))
  q = _seg(y2, src, dst)
  if isinstance(q, (list, tuple)):
    q = q[0]
  out = _post(q.reshape(_NC, _G, _R, _D), inv, r2)
  return out[:_N]
